# baseline (speedup 1.0000x reference)
import functools

import jax
import jax.numpy as jnp
from jax.experimental import pallas as pl
from jax.experimental.pallas import tpu as pltpu

_LAMBDA_U = 0.01
_COEF_S = 0.1


def _loss_kernel(coarse_ref, fine_ref, beta_ref, sig_ref, rgbs_ref, mask_ref,
                 out_ref, acc_ref, *, total_sig):
    i = pl.program_id(0)
    n_blocks = pl.num_programs(0)

    @pl.when(i == 0)
    def _init():
        for k in range(5):
            acc_ref[k] = 0.0

    mask = mask_ref[...]
    beta = beta_ref[...]
    rgbs = rgbs_ref[...]
    cd = coarse_ref[...] - rgbs
    fd = fine_ref[...] - rgbs
    acc_ref[0] += jnp.sum(cd * cd * mask)
    acc_ref[1] += jnp.sum(
        jnp.sum(fd * fd, axis=1, keepdims=True) * (mask / (2.0 * beta * beta)))
    acc_ref[2] += jnp.sum(jnp.log(beta) * mask)
    acc_ref[3] += jnp.sum(sig_ref[...])
    acc_ref[4] += jnp.sum(mask)

    @pl.when(i == n_blocks - 1)
    def _fin():
        inv = 1.0 / (acc_ref[4] + 1e-20)
        out_ref[0] = 0.5 * acc_ref[0] * inv
        out_ref[1] = acc_ref[1] * inv
        out_ref[2] = 3.0 + acc_ref[2] * inv
        out_ref[3] = _COEF_S * _LAMBDA_U * acc_ref[3] / total_sig


def kernel(rgb_coarse, rgb_fine_combined, beta, transient_sigmas, rgbs, ray_mask):
    n, s = transient_sigmas.shape
    block = 2048
    grid = n // block
    beta2 = beta.reshape(n, 1)

    out = pl.pallas_call(
        functools.partial(_loss_kernel, total_sig=float(n * s)),
        grid=(grid,),
        in_specs=[
            pl.BlockSpec((block, 3), lambda i: (i, 0)),
            pl.BlockSpec((block, 3), lambda i: (i, 0)),
            pl.BlockSpec((block, 1), lambda i: (i, 0)),
            pl.BlockSpec((block, s), lambda i: (i, 0)),
            pl.BlockSpec((block, 3), lambda i: (i, 0)),
            pl.BlockSpec((block, 1), lambda i: (i, 0)),
        ],
        out_specs=pl.BlockSpec(memory_space=pltpu.SMEM),
        out_shape=jax.ShapeDtypeStruct((4,), jnp.float32),
        scratch_shapes=[pltpu.SMEM((5,), jnp.float32)],
    )(rgb_coarse, rgb_fine_combined, beta2, transient_sigmas, rgbs, ray_mask)
    return out
